# Initial kernel scaffold; baseline (speedup 1.0000x reference)
import jax
import jax.numpy as jnp
from jax.experimental import pallas as pl


def kernel(feats, edge_dict, W, b):
    raise NotImplementedError("write your pallas kernel here")


# speedup vs baseline: 1.1018x; 1.1018x over previous
import functools

import jax
import jax.numpy as jnp
from jax import lax
from jax.experimental import pallas as pl
from jax.experimental.pallas import tpu as pltpu
from jax.experimental.pallas import tpu_sc as plsc

N = 10000
DEG = 16
D = 256
LANES = 16

NW = 32
NPW = 320
NPAD = NW * NPW
CHUNK = 8
NCHUNKS = NPW // CHUNK

MM_BLOCK = 1000


def _mm_body(f_ref, wt_ref, b_ref, o_ref):
    acc = jnp.dot(f_ref[...], wt_ref[...], preferred_element_type=jnp.float32)
    o_ref[...] = jnp.maximum(acc + b_ref[...], 0.0)


def _linear_relu(feats, wt, b_row):
    return pl.pallas_call(
        _mm_body,
        grid=(N // MM_BLOCK,),
        in_specs=[
            pl.BlockSpec((MM_BLOCK, D), lambda i: (i, 0)),
            pl.BlockSpec((D, D), lambda i: (0, 0)),
            pl.BlockSpec((1, D), lambda i: (0, 0)),
        ],
        out_specs=pl.BlockSpec((MM_BLOCK, D), lambda i: (i, 0)),
        out_shape=jax.ShapeDtypeStruct((N, D), jnp.float32),
    )(feats, wt, b_row)


def _agg_body(x_hbm, edge_hbm, out_hbm, idx_v, rows_v, out_v, sem):
    wid = lax.axis_index("s") * 2 + lax.axis_index("c")
    pltpu.sync_copy(edge_hbm.at[wid], idx_v)

    def chunk_body(g, carry):
        pltpu.async_copy(x_hbm.at[idx_v.at[g]], rows_v, sem).wait()

        def node_body(n, carry2):
            base = n * DEG
            for k in range(D // LANES):
                sl = pl.ds(k * LANES, LANES)
                acc = rows_v[base, sl]
                for j in range(1, DEG):
                    acc = acc + rows_v[base + j, sl]
                out_v[n, sl] = acc * (1.0 / DEG)
            return carry2

        lax.fori_loop(0, CHUNK, node_body, 0)
        pltpu.sync_copy(
            out_v, out_hbm.at[pl.ds(wid * NPW + g * CHUNK, CHUNK)]
        )
        return carry

    lax.fori_loop(0, NCHUNKS, chunk_body, 0)


def _aggregate(x, edge_r):
    mesh = plsc.VectorSubcoreMesh(core_axis_name="c", subcore_axis_name="s")
    agg = functools.partial(
        pl.kernel,
        out_type=jax.ShapeDtypeStruct((NPAD, D), jnp.float32),
        mesh=mesh,
        scratch_types=[
            pltpu.VMEM((NCHUNKS, CHUNK * DEG), jnp.int32),
            pltpu.VMEM((CHUNK * DEG, D), jnp.float32),
            pltpu.VMEM((CHUNK, D), jnp.float32),
            pltpu.SemaphoreType.DMA,
        ],
    )(_agg_body)
    return agg(x, edge_r)


def kernel(feats, edge_dict, W, b):
    wt = W.T
    b_row = b.reshape(1, D)
    x = _linear_relu(feats, wt, b_row)

    edge_pad = jnp.concatenate(
        [edge_dict, jnp.zeros((NPAD - N, DEG), jnp.int32)], axis=0
    )
    edge_r = edge_pad.reshape(NW, NCHUNKS, CHUNK * DEG)

    pooled = _aggregate(x, edge_r)
    return pooled[:N]


# speedup vs baseline: 1.4056x; 1.2757x over previous
import functools

import jax
import jax.numpy as jnp
from jax import lax
from jax.experimental import pallas as pl
from jax.experimental.pallas import tpu as pltpu
from jax.experimental.pallas import tpu_sc as plsc

N = 10000
DEG = 16
D = 256
LANES = 16

NW = 32
NPW = 320
NPAD = NW * NPW
CHUNK = 8
NCHUNKS = NPW // CHUNK

MM_BLOCK = 1000


def _mm_body(f_ref, wt_ref, b_ref, o_ref):
    acc = jnp.dot(f_ref[...], wt_ref[...], preferred_element_type=jnp.float32)
    o_ref[...] = jnp.maximum(acc + b_ref[...], 0.0)


def _linear_relu(feats, wt, b_row):
    return pl.pallas_call(
        _mm_body,
        grid=(N // MM_BLOCK,),
        in_specs=[
            pl.BlockSpec((MM_BLOCK, D), lambda i: (i, 0)),
            pl.BlockSpec((D, D), lambda i: (0, 0)),
            pl.BlockSpec((1, D), lambda i: (0, 0)),
        ],
        out_specs=pl.BlockSpec((MM_BLOCK, D), lambda i: (i, 0)),
        out_shape=jax.ShapeDtypeStruct((N, D), jnp.float32),
    )(feats, wt, b_row)


def _agg_body(x_hbm, edge_hbm, out_hbm, idx_v, rows0, rows1, out_v, sem0, sem1):
    wid = lax.axis_index("s") * 2 + lax.axis_index("c")
    pltpu.sync_copy(edge_hbm.at[wid], idx_v)

    bufs = (rows0, rows1)
    sems = (sem0, sem1)

    pltpu.async_copy(x_hbm.at[idx_v.at[0]], rows0, sem0)
    pltpu.async_copy(x_hbm.at[idx_v.at[1]], rows1, sem1)

    def accum(rbuf, g):
        def node_body(n, carry2):
            base = n * DEG
            for k in range(D // LANES):
                sl = pl.ds(k * LANES, LANES)
                v = [rbuf[base + j, sl] for j in range(DEG)]
                while len(v) > 1:
                    v = [v[2 * i] + v[2 * i + 1] for i in range(len(v) // 2)]
                out_v[n, sl] = v[0] * (1.0 / DEG)
            return carry2

        lax.fori_loop(0, CHUNK, node_body, 0)
        pltpu.sync_copy(
            out_v, out_hbm.at[pl.ds(wid * NPW + g * CHUNK, CHUNK)]
        )

    def pair_body(p, carry):
        for b in range(2):
            g = p * 2 + b
            rbuf, sem = bufs[b], sems[b]
            pltpu.make_async_copy(x_hbm.at[idx_v.at[g]], rbuf, sem).wait()
            accum(rbuf, g)

            @pl.when(g + 2 < NCHUNKS)
            def _():
                pltpu.async_copy(x_hbm.at[idx_v.at[g + 2]], rbuf, sem)

        return carry

    lax.fori_loop(0, NCHUNKS // 2, pair_body, 0)


def _aggregate(x, edge_r):
    mesh = plsc.VectorSubcoreMesh(core_axis_name="c", subcore_axis_name="s")
    agg = functools.partial(
        pl.kernel,
        out_type=jax.ShapeDtypeStruct((NPAD, D), jnp.float32),
        mesh=mesh,
        scratch_types=[
            pltpu.VMEM((NCHUNKS, CHUNK * DEG), jnp.int32),
            pltpu.VMEM((CHUNK * DEG, D), jnp.float32),
            pltpu.VMEM((CHUNK * DEG, D), jnp.float32),
            pltpu.VMEM((CHUNK, D), jnp.float32),
            pltpu.SemaphoreType.DMA,
            pltpu.SemaphoreType.DMA,
        ],
    )(_agg_body)
    return agg(x, edge_r)


def kernel(feats, edge_dict, W, b):
    wt = W.T
    b_row = b.reshape(1, D)
    x = _linear_relu(feats, wt, b_row)

    edge_pad = jnp.concatenate(
        [edge_dict, jnp.zeros((NPAD - N, DEG), jnp.int32)], axis=0
    )
    edge_r = edge_pad.reshape(NW, NCHUNKS, CHUNK * DEG)

    pooled = _aggregate(x, edge_r)
    return pooled[:N]
